# baseline (device time: 1213121 ns/iter reference)
import jax
import jax.numpy as jnp
from jax import lax
from jax.experimental import pallas as pl
from jax.experimental.pallas import tpu as pltpu

K = 32


def kernel(x):
    return _passthrough_copy(_rs_kernel(x))


def _rs_kernel(x):
    _, m, n = x.shape
    n_out = n // 2
    m_half = m // 2
    c = m_half // K

    def body(x_ref, out_ref, recv_buf, local_buf,
             x_send_sem, x_recv_sems, y_send_sem, y_recv_sem,
             in_sem, out_sem):
        my_x = lax.axis_index("x")
        my_y = lax.axis_index("y")
        row0 = my_y * m_half
        my_col0 = my_x * n_out
        peer_col0 = (1 - my_x) * n_out

        barrier_sem = pltpu.get_barrier_semaphore()
        pl.semaphore_signal(barrier_sem, inc=1, device_id=(1 - my_x, my_y),
                            device_id_type=pl.DeviceIdType.MESH)
        pl.semaphore_signal(barrier_sem, inc=1, device_id=(my_x, 1 - my_y),
                            device_id_type=pl.DeviceIdType.MESH)
        pl.semaphore_wait(barrier_sem, 2)

        cp_in = pltpu.make_async_copy(
            x_ref.at[0, pl.ds(row0, m_half), pl.ds(my_col0, n_out)],
            local_buf, in_sem)
        cp_in.start()

        x_rdmas = []
        for i in range(K):
            r = pltpu.make_async_remote_copy(
                src_ref=x_ref.at[0, pl.ds(row0 + i * c, c),
                                 pl.ds(peer_col0, n_out)],
                dst_ref=recv_buf.at[pl.ds(i * c, c), :],
                send_sem=x_send_sem,
                recv_sem=x_recv_sems.at[i],
                device_id=(1 - my_x, my_y),
                device_id_type=pl.DeviceIdType.MESH,
            )
            r.start()
            x_rdmas.append(r)

        cp_in.wait()

        y_rdmas, out_cps = [], []
        for i in range(K):
            x_rdmas[i].wait_recv()
            recv_buf[pl.ds(i * c, c), :] = (
                recv_buf[pl.ds(i * c, c), :] + local_buf[pl.ds(i * c, c), :])
            ry = pltpu.make_async_remote_copy(
                src_ref=recv_buf.at[pl.ds(i * c, c), :],
                dst_ref=out_ref.at[pl.ds(row0 + i * c, c), :],
                send_sem=y_send_sem,
                recv_sem=y_recv_sem,
                device_id=(my_x, 1 - my_y),
                device_id_type=pl.DeviceIdType.MESH,
            )
            ry.start()
            y_rdmas.append(ry)
            cp = pltpu.make_async_copy(
                recv_buf.at[pl.ds(i * c, c), :],
                out_ref.at[pl.ds(row0 + i * c, c), :], out_sem)
            cp.start()
            out_cps.append(cp)

        for i in range(K):
            x_rdmas[i].wait_send()
            y_rdmas[i].wait()
            out_cps[i].wait()

    return pl.pallas_call(
        body,
        out_shape=jax.ShapeDtypeStruct((m, n_out), x.dtype),
        in_specs=[pl.BlockSpec(memory_space=pl.ANY)],
        out_specs=pl.BlockSpec(memory_space=pl.ANY),
        scratch_shapes=[
            pltpu.VMEM((m_half, n_out), x.dtype),
            pltpu.VMEM((m_half, n_out), x.dtype),
            pltpu.SemaphoreType.DMA,
            pltpu.SemaphoreType.DMA((K,)),
            pltpu.SemaphoreType.DMA,
            pltpu.SemaphoreType.DMA,
            pltpu.SemaphoreType.DMA,
            pltpu.SemaphoreType.DMA,
        ],
        compiler_params=pltpu.CompilerParams(
            collective_id=0,
            vmem_limit_bytes=40 * 1024 * 1024,
        ),
    )(x)


def _passthrough_copy(out):
    m, n_out = out.shape
    parts = 8
    rows = m // parts

    def body(src_ref, o_ref, sems):
        cps = []
        for i in range(parts):
            cp = pltpu.make_async_copy(
                src_ref.at[pl.ds(i * rows, rows), :],
                o_ref.at[pl.ds(i * rows, rows), :],
                sems.at[i])
            cp.start()
            cps.append(cp)
        for cp in cps:
            cp.wait()

    return pl.pallas_call(
        body,
        out_shape=jax.ShapeDtypeStruct((m, n_out), out.dtype),
        in_specs=[pl.BlockSpec(memory_space=pl.ANY)],
        out_specs=pl.BlockSpec(memory_space=pl.ANY),
        scratch_shapes=[pltpu.SemaphoreType.DMA((parts,))],
    )(out)


# device time: 214564 ns/iter; 5.6539x vs baseline; 5.6539x over previous
import jax
import jax.numpy as jnp
from jax import lax
from jax.experimental import pallas as pl
from jax.experimental.pallas import tpu as pltpu

K = 64


def kernel(x):
    _, m, n = x.shape
    n_out = n // 2
    m_half = m // 2
    c = m_half // K

    def body(x_ref, out_ref, recv_buf, local_buf,
             x_send_sem, x_recv_sems, y_send_sem, y_recv_sem,
             in_sem, out_sem):
        my_x = lax.axis_index("x")
        my_y = lax.axis_index("y")
        row0 = my_y * m_half
        my_col0 = my_x * n_out
        peer_col0 = (1 - my_x) * n_out

        barrier_sem = pltpu.get_barrier_semaphore()
        pl.semaphore_signal(barrier_sem, inc=1, device_id=(1 - my_x, my_y),
                            device_id_type=pl.DeviceIdType.MESH)
        pl.semaphore_signal(barrier_sem, inc=1, device_id=(my_x, 1 - my_y),
                            device_id_type=pl.DeviceIdType.MESH)
        pl.semaphore_wait(barrier_sem, 2)

        cp_in = pltpu.make_async_copy(
            x_ref.at[0, pl.ds(row0, m_half), pl.ds(my_col0, n_out)],
            local_buf, in_sem)
        cp_in.start()

        x_rdmas = []
        for i in range(K):
            r = pltpu.make_async_remote_copy(
                src_ref=x_ref.at[0, pl.ds(row0 + i * c, c),
                                 pl.ds(peer_col0, n_out)],
                dst_ref=recv_buf.at[pl.ds(i * c, c), :],
                send_sem=x_send_sem,
                recv_sem=x_recv_sems.at[i],
                device_id=(1 - my_x, my_y),
                device_id_type=pl.DeviceIdType.MESH,
            )
            r.start()
            x_rdmas.append(r)

        cp_in.wait()

        y_rdmas, out_cps = [], []
        for i in range(K):
            x_rdmas[i].wait_recv()
            recv_buf[pl.ds(i * c, c), :] = (
                recv_buf[pl.ds(i * c, c), :] + local_buf[pl.ds(i * c, c), :])
            ry = pltpu.make_async_remote_copy(
                src_ref=recv_buf.at[pl.ds(i * c, c), :],
                dst_ref=out_ref.at[pl.ds(row0 + i * c, c), :],
                send_sem=y_send_sem,
                recv_sem=y_recv_sem,
                device_id=(my_x, 1 - my_y),
                device_id_type=pl.DeviceIdType.MESH,
            )
            ry.start()
            y_rdmas.append(ry)
            cp = pltpu.make_async_copy(
                recv_buf.at[pl.ds(i * c, c), :],
                out_ref.at[pl.ds(row0 + i * c, c), :], out_sem)
            cp.start()
            out_cps.append(cp)

        for i in range(K):
            x_rdmas[i].wait_send()
            y_rdmas[i].wait()
            out_cps[i].wait()

    return pl.pallas_call(
        body,
        out_shape=jax.ShapeDtypeStruct((m, n_out), x.dtype),
        in_specs=[pl.BlockSpec(memory_space=pl.ANY)],
        out_specs=pl.BlockSpec(memory_space=pl.ANY),
        scratch_shapes=[
            pltpu.VMEM((m_half, n_out), x.dtype),
            pltpu.VMEM((m_half, n_out), x.dtype),
            pltpu.SemaphoreType.DMA,
            pltpu.SemaphoreType.DMA((K,)),
            pltpu.SemaphoreType.DMA,
            pltpu.SemaphoreType.DMA,
            pltpu.SemaphoreType.DMA,
            pltpu.SemaphoreType.DMA,
        ],
        compiler_params=pltpu.CompilerParams(
            collective_id=0,
            vmem_limit_bytes=40 * 1024 * 1024,
        ),
    )(x)
